# baseline (speedup 1.0000x reference)
import functools

import jax
import jax.numpy as jnp
from jax import lax
from jax.experimental import pallas as pl
from jax.experimental.pallas import tpu as pltpu
from jax.experimental.pallas import tpu_sc as plsc

N, E, D, DE, H = 10000, 320000, 128, 16, 128
NC, NS = 2, 16
NW = NC * NS
EW = E // NW
C = 400
KCH = EW // C
CS = 80
KS = EW // CS
N_PAD = 10240
RPT = N_PAD // NS

_mesh = plsc.VectorSubcoreMesh(core_axis_name="c", subcore_axis_name="s")


def _prep_body(x_ref, pos4_ref, w1a_ref, w1b_ref, w1d_ref, ta_ref, tb_ref):
    xb = x_ref[...]
    na = jnp.sum(pos4_ref[...] * pos4_ref[...], axis=1, keepdims=True)
    nterm = na * w1d_ref[...]
    ta_ref[...] = jnp.dot(xb, w1a_ref[...], preferred_element_type=jnp.float32) + nterm
    tb_ref[...] = jnp.dot(xb, w1b_ref[...], preferred_element_type=jnp.float32) + nterm


_BN = 1000

_prep_call = pl.pallas_call(
    _prep_body,
    grid=(N // _BN,),
    in_specs=[
        pl.BlockSpec((_BN, D), lambda i: (i, 0)),
        pl.BlockSpec((_BN, 16), lambda i: (i, 0)),
        pl.BlockSpec((D, H), lambda i: (0, 0)),
        pl.BlockSpec((D, H), lambda i: (0, 0)),
        pl.BlockSpec((1, H), lambda i: (0, 0)),
    ],
    out_specs=[
        pl.BlockSpec((_BN, H), lambda i: (i, 0)),
        pl.BlockSpec((_BN, H), lambda i: (i, 0)),
    ],
    out_shape=[
        jax.ShapeDtypeStruct((N, H), jnp.float32),
        jax.ShapeDtypeStruct((N, H), jnp.float32),
    ],
)


def _gather_body(ta_h, tb_h, px_h, py_h, pz_h, src_h, dst_h, g_h, dots_h,
                 idx_a, idx_b, gbuf_a, gbuf_b, pos_a, pos_b, dbuf_a, dbuf_b,
                 sem_ia, sem_ib, sem_ga, sem_gb, sem_pa, sem_pb,
                 sem_wa, sem_wb):
    c = lax.axis_index("c")
    s = lax.axis_index("s")
    base_w = (c * NS + s) * EW
    A = (idx_a, gbuf_a, pos_a, dbuf_a, sem_ia, sem_ga, sem_pa, sem_wa)
    B = (idx_b, gbuf_b, pos_b, dbuf_b, sem_ib, sem_gb, sem_pb, sem_wb)

    def load_idx(g, sl):
        idx, _, _, _, sem_i, _, _, _ = sl
        pltpu.async_copy(src_h.at[pl.ds(base_w + g * C, C)], idx[0], sem_i)
        pltpu.async_copy(dst_h.at[pl.ds(base_w + g * C, C)], idx[1], sem_i)

    def drain_idx(sl):
        idx, _, _, _, sem_i, _, _, _ = sl
        pltpu.make_async_copy(src_h.at[pl.ds(0, C)], idx[0], sem_i).wait()
        pltpu.make_async_copy(src_h.at[pl.ds(0, C)], idx[1], sem_i).wait()

    def front(g, sl):
        idx, gbuf, posb, _, _, sem_g, sem_p, _ = sl
        src = idx[0]
        dst = idx[1]
        pltpu.async_copy(ta_h.at[dst], gbuf, sem_g)
        for t_h, ix, buf in ((px_h, dst, posb[0]), (py_h, dst, posb[1]),
                             (pz_h, dst, posb[2]), (px_h, src, posb[3]),
                             (py_h, src, posb[4]), (pz_h, src, posb[5])):
            pltpu.async_copy(t_h.at[ix], buf, sem_p)

    def drain_g(sl):
        _, gbuf, _, _, _, sem_g, _, _ = sl
        pltpu.make_async_copy(g_h.at[pl.ds(0, C)], gbuf, sem_g).wait()

    def drain_p(sl):
        _, _, posb, _, _, _, sem_p, _ = sl
        for b in posb:
            pltpu.make_async_copy(px_h.at[pl.ds(0, C)], b, sem_p).wait()

    def drain_w(sl):
        _, gbuf, _, dbuf, _, _, _, sem_w = sl
        pltpu.make_async_copy(gbuf, g_h.at[pl.ds(0, C)], sem_w).wait()
        pltpu.make_async_copy(dbuf, dots_h.at[pl.ds(0, C)], sem_w).wait()

    load_idx(0, A)
    load_idx(1, B)
    drain_idx(A)
    front(0, A)

    def body(o, carry):
        ga = 2 * o

        drain_idx(B)

        @pl.when(o > 0)
        def _():
            drain_w(B)

        front(ga + 1, B)
        drain_g(A)
        pltpu.async_copy(tb_h.at[idx_a[0]], gbuf_a, sem_ga, add=True)
        drain_p(A)

        def dot_a(t, carry2):
            t16 = pl.ds(t * 16, 16)
            dbuf_a[t16] = (pos_a[0][t16] * pos_a[3][t16]
                           + pos_a[1][t16] * pos_a[4][t16]
                           + pos_a[2][t16] * pos_a[5][t16])
            return carry2

        lax.fori_loop(0, C // 16, dot_a, 0)
        drain_g(A)
        load_idx(ga + 2, A)
        pltpu.async_copy(gbuf_a, g_h.at[pl.ds(base_w + ga * C, C)], sem_wa)
        pltpu.async_copy(dbuf_a, dots_h.at[pl.ds(base_w + ga * C, C)], sem_wa)

        drain_idx(A)
        drain_w(A)
        front(ga + 2, A)
        drain_g(B)
        pltpu.async_copy(tb_h.at[idx_b[0]], gbuf_b, sem_gb, add=True)
        drain_p(B)

        def dot_b(t, carry2):
            t16 = pl.ds(t * 16, 16)
            dbuf_b[t16] = (pos_b[0][t16] * pos_b[3][t16]
                           + pos_b[1][t16] * pos_b[4][t16]
                           + pos_b[2][t16] * pos_b[5][t16])
            return carry2

        lax.fori_loop(0, C // 16, dot_b, 0)
        drain_g(B)

        @pl.when(o < (KCH - 1) // 2 - 1)
        def _():
            load_idx(ga + 3, B)

        pltpu.async_copy(gbuf_b, g_h.at[pl.ds(base_w + (ga + 1) * C, C)],
                         sem_wb)
        pltpu.async_copy(dbuf_b, dots_h.at[pl.ds(base_w + (ga + 1) * C, C)],
                         sem_wb)
        return carry

    lax.fori_loop(0, (KCH - 1) // 2, body, 0)

    g_last = KCH - 1
    drain_g(A)
    pltpu.async_copy(tb_h.at[idx_a[0]], gbuf_a, sem_ga, add=True)
    drain_p(A)

    def dot_tail(t, carry2):
        t16 = pl.ds(t * 16, 16)
        dbuf_a[t16] = (pos_a[0][t16] * pos_a[3][t16]
                       + pos_a[1][t16] * pos_a[4][t16]
                       + pos_a[2][t16] * pos_a[5][t16])
        return carry2

    lax.fori_loop(0, C // 16, dot_tail, 0)
    drain_g(A)
    drain_w(B)
    pltpu.async_copy(gbuf_a, g_h.at[pl.ds(base_w + g_last * C, C)], sem_wa)
    pltpu.async_copy(dbuf_a, dots_h.at[pl.ds(base_w + g_last * C, C)], sem_wa)
    drain_w(A)


_gather_call = functools.partial(
    pl.kernel,
    out_type=[
        jax.ShapeDtypeStruct((E, H), jnp.float32),
        jax.ShapeDtypeStruct((E,), jnp.float32),
    ],
    mesh=_mesh,
    scratch_types=[
        [pltpu.VMEM((C,), jnp.int32)] * 2,
        [pltpu.VMEM((C,), jnp.int32)] * 2,
        pltpu.VMEM((C, H), jnp.float32),
        pltpu.VMEM((C, H), jnp.float32),
        [pltpu.VMEM((C,), jnp.float32)] * 6,
        [pltpu.VMEM((C,), jnp.float32)] * 6,
        pltpu.VMEM((C,), jnp.float32),
        pltpu.VMEM((C,), jnp.float32),
        pltpu.SemaphoreType.DMA,
        pltpu.SemaphoreType.DMA,
        pltpu.SemaphoreType.DMA,
        pltpu.SemaphoreType.DMA,
        pltpu.SemaphoreType.DMA,
        pltpu.SemaphoreType.DMA,
        pltpu.SemaphoreType.DMA,
        pltpu.SemaphoreType.DMA,
    ],
)(_gather_body)


_BE = 1000


def _edge_mlp_body(g_ref, ea_ref, dots_ref, w1c_ref, w2_ref, b1_ref,
                   b2_ref, w1d_ref, out_ref):
    d = jnp.transpose(dots_ref[...].reshape(1, _BE), (1, 0))
    pre = (g_ref[...]
           + jnp.dot(ea_ref[...], w1c_ref[...], preferred_element_type=jnp.float32)
           + b1_ref[...]
           - 2.0 * d * w1d_ref[...])
    m1 = jnp.maximum(pre, 0.0).astype(jnp.bfloat16)
    m2 = jnp.dot(m1, w2_ref[...], preferred_element_type=jnp.float32) + b2_ref[...]
    out_ref[...] = jnp.maximum(m2, 0.0)


_edge_mlp_call = pl.pallas_call(
    _edge_mlp_body,
    grid=(E // _BE,),
    in_specs=[
        pl.BlockSpec((_BE, H), lambda i: (i, 0)),
        pl.BlockSpec((_BE, DE), lambda i: (i, 0)),
        pl.BlockSpec((1, 1, _BE), lambda i: (i, 0, 0)),
        pl.BlockSpec((DE, H), lambda i: (0, 0)),
        pl.BlockSpec((H, H), lambda i: (0, 0)),
        pl.BlockSpec((1, H), lambda i: (0, 0)),
        pl.BlockSpec((1, H), lambda i: (0, 0)),
        pl.BlockSpec((1, H), lambda i: (0, 0)),
    ],
    out_specs=pl.BlockSpec((_BE, H), lambda i: (i, 0)),
    out_shape=jax.ShapeDtypeStruct((E, H), jnp.float32),
)


def _scatter_body(m2_h, dst_h, zeros_h, out_h, idx0, idx1, mb0, mb1, aggr_sp,
                  sem_i, sem_m):
    c = lax.axis_index("c")
    s = lax.axis_index("s")
    row0 = s * RPT
    pltpu.sync_copy(zeros_h.at[pl.ds(row0, RPT)],
                    aggr_sp.at[pl.ds(row0, RPT)])
    base = (c * NS + s) * EW
    plsc.subcore_barrier()

    def start(g, idxb, mbuf):
        off = base + g * CS
        pltpu.async_copy(dst_h.at[pl.ds(off, CS)], idxb, sem_i)
        pltpu.async_copy(m2_h.at[pl.ds(off, CS)], mbuf, sem_m)

    def drain():
        pltpu.make_async_copy(dst_h.at[pl.ds(base, CS)], idx0, sem_i).wait()
        pltpu.make_async_copy(m2_h.at[pl.ds(base, CS)], mb0, sem_m).wait()

    def scatter(idxb, mbuf):
        pltpu.sync_copy(mbuf, aggr_sp.at[idxb], add=True)

    start(0, idx0, mb0)

    def outer(o, carry):
        g0 = 2 * o
        drain()
        start(g0 + 1, idx1, mb1)
        scatter(idx0, mb0)
        drain()
        start(g0 + 2, idx0, mb0)
        scatter(idx1, mb1)
        return carry

    lax.fori_loop(0, (KS - 1) // 2, outer, 0)
    drain()
    scatter(idx0, mb0)
    plsc.subcore_barrier()
    pltpu.sync_copy(aggr_sp.at[pl.ds(row0, RPT)],
                    out_h.at[c, pl.ds(row0, RPT)])


_scatter_call = functools.partial(
    pl.kernel,
    out_type=jax.ShapeDtypeStruct((NC, N_PAD, H), jnp.float32),
    mesh=_mesh,
    scratch_types=[
        pltpu.VMEM((CS,), jnp.int32),
        pltpu.VMEM((CS,), jnp.int32),
        pltpu.VMEM((CS, H), jnp.float32),
        pltpu.VMEM((CS, H), jnp.float32),
        pltpu.VMEM_SHARED((N_PAD, H), jnp.float32),
        pltpu.SemaphoreType.DMA,
        pltpu.SemaphoreType.DMA,
    ],
)(_scatter_body)


def _node_mlp_body(x_ref, p_ref, wn1a_ref, wn1b_ref, bn1_ref, wn2_ref, bn2_ref,
                   out_ref):
    aggr = p_ref[0] + p_ref[1]
    h = jnp.maximum(
        jnp.dot(x_ref[...], wn1a_ref[...], preferred_element_type=jnp.float32)
        + jnp.dot(aggr, wn1b_ref[...], preferred_element_type=jnp.float32)
        + bn1_ref[...], 0.0)
    out_ref[...] = (jnp.dot(h, wn2_ref[...], preferred_element_type=jnp.float32)
                    + bn2_ref[...])


_node_mlp_call = pl.pallas_call(
    _node_mlp_body,
    grid=(N // _BN,),
    in_specs=[
        pl.BlockSpec((_BN, D), lambda i: (i, 0)),
        pl.BlockSpec((NC, _BN, H), lambda i: (0, i, 0)),
        pl.BlockSpec((D, H), lambda i: (0, 0)),
        pl.BlockSpec((H, H), lambda i: (0, 0)),
        pl.BlockSpec((1, H), lambda i: (0, 0)),
        pl.BlockSpec((H, D), lambda i: (0, 0)),
        pl.BlockSpec((1, D), lambda i: (0, 0)),
    ],
    out_specs=pl.BlockSpec((_BN, D), lambda i: (i, 0)),
    out_shape=jax.ShapeDtypeStruct((N, D), jnp.float32),
)


def kernel(x, pos, edge_index, edge_attr, W1, b1, W2, b2, Wn1, bn1, Wn2, bn2):
    ei = edge_index.astype(jnp.int32)
    dst = ei[1]
    pos4 = jnp.pad(pos, ((0, 0), (0, 13)))
    px, py, pz = pos[:, 0], pos[:, 1], pos[:, 2]
    W1a = W1[:D]
    W1b = W1[D:2 * D]
    W1c = W1[2 * D:2 * D + DE]
    w1d = W1[2 * D + DE][None, :]

    Ta, Tb = _prep_call(x, pos4, W1a, W1b, w1d)
    G, dots = _gather_call(Ta, Tb, px, py, pz, ei[0], ei[1])
    dots3d = dots.reshape(E // _BE, 1, _BE)
    m2 = _edge_mlp_call(G, edge_attr, dots3d, W1c, W2.astype(jnp.bfloat16),
                        b1[None], b2[None], w1d)
    partials = _scatter_call(m2, dst, jnp.zeros((N_PAD, H), jnp.float32))
    out = _node_mlp_call(x, partials, Wn1[:D], Wn1[D:], bn1[None], Wn2, bn2[None])
    return out


# speedup vs baseline: 1.0156x; 1.0156x over previous
import functools

import jax
import jax.numpy as jnp
from jax import lax
from jax.experimental import pallas as pl
from jax.experimental.pallas import tpu as pltpu
from jax.experimental.pallas import tpu_sc as plsc

N, E, D, DE, H = 10000, 320000, 128, 16, 128
NC, NS = 2, 16
NW = NC * NS
EW = E // NW
C = 400
KCH = EW // C
CS = 80
KS = EW // CS
N_PAD = 10240
RPT = N_PAD // NS

_mesh = plsc.VectorSubcoreMesh(core_axis_name="c", subcore_axis_name="s")


def _prep_body(x_ref, pos4_ref, w1a_ref, w1b_ref, w1d_ref, ta_ref, tb_ref):
    xb = x_ref[...]
    na = jnp.sum(pos4_ref[...] * pos4_ref[...], axis=1, keepdims=True)
    nterm = na * w1d_ref[...]
    ta_ref[...] = jnp.dot(xb, w1a_ref[...], preferred_element_type=jnp.float32) + nterm
    tb_ref[...] = jnp.dot(xb, w1b_ref[...], preferred_element_type=jnp.float32) + nterm


_BN = 1000

_prep_call = pl.pallas_call(
    _prep_body,
    grid=(N // _BN,),
    in_specs=[
        pl.BlockSpec((_BN, D), lambda i: (i, 0)),
        pl.BlockSpec((_BN, 16), lambda i: (i, 0)),
        pl.BlockSpec((D, H), lambda i: (0, 0)),
        pl.BlockSpec((D, H), lambda i: (0, 0)),
        pl.BlockSpec((1, H), lambda i: (0, 0)),
    ],
    out_specs=[
        pl.BlockSpec((_BN, H), lambda i: (i, 0)),
        pl.BlockSpec((_BN, H), lambda i: (i, 0)),
    ],
    out_shape=[
        jax.ShapeDtypeStruct((N, H), jnp.float32),
        jax.ShapeDtypeStruct((N, H), jnp.float32),
    ],
)


def _gather_body(ta_h, tb_h, px_h, py_h, pz_h, src_h, dst_h, g_h, dots_h,
                 idx_a, idx_b, gbuf_a, gbuf_b, pos_a, pos_b, dbuf_a, dbuf_b,
                 sem_ia, sem_ib, sem_ga, sem_gb, sem_pa, sem_pb,
                 sem_wa, sem_wb):
    c = lax.axis_index("c")
    s = lax.axis_index("s")
    base_w = (c * NS + s) * EW
    A = (idx_a, gbuf_a, pos_a, dbuf_a, sem_ia, sem_ga, sem_pa, sem_wa)
    B = (idx_b, gbuf_b, pos_b, dbuf_b, sem_ib, sem_gb, sem_pb, sem_wb)

    def load_idx(g, sl):
        idx, _, _, _, sem_i, _, _, _ = sl
        pltpu.async_copy(src_h.at[pl.ds(base_w + g * C, C)], idx[0], sem_i)
        pltpu.async_copy(dst_h.at[pl.ds(base_w + g * C, C)], idx[1], sem_i)

    def drain_idx(sl):
        idx, _, _, _, sem_i, _, _, _ = sl
        pltpu.make_async_copy(src_h.at[pl.ds(0, C)], idx[0], sem_i).wait()
        pltpu.make_async_copy(src_h.at[pl.ds(0, C)], idx[1], sem_i).wait()

    def front(g, sl):
        idx, gbuf, posb, _, _, sem_g, sem_p, _ = sl
        src = idx[0]
        dst = idx[1]
        pltpu.async_copy(ta_h.at[dst], gbuf, sem_g)
        for t_h, ix, buf in ((px_h, dst, posb[0]), (py_h, dst, posb[1]),
                             (pz_h, dst, posb[2]), (px_h, src, posb[3]),
                             (py_h, src, posb[4]), (pz_h, src, posb[5])):
            pltpu.async_copy(t_h.at[ix], buf, sem_p)

    def drain_g(sl):
        _, gbuf, _, _, _, sem_g, _, _ = sl
        pltpu.make_async_copy(g_h.at[pl.ds(0, C)], gbuf, sem_g).wait()

    def drain_p(sl):
        _, _, posb, _, _, _, sem_p, _ = sl
        for b in posb:
            pltpu.make_async_copy(px_h.at[pl.ds(0, C)], b, sem_p).wait()

    def drain_w(sl):
        _, gbuf, _, dbuf, _, _, _, sem_w = sl
        pltpu.make_async_copy(gbuf, g_h.at[pl.ds(0, C)], sem_w).wait()
        pltpu.make_async_copy(dbuf, dots_h.at[pl.ds(0, C)], sem_w).wait()

    load_idx(0, A)
    load_idx(1, B)
    drain_idx(A)
    front(0, A)

    def body(o, carry):
        ga = 2 * o

        drain_idx(B)

        @pl.when(o > 0)
        def _():
            drain_w(B)

        front(ga + 1, B)
        drain_g(A)
        pltpu.async_copy(tb_h.at[idx_a[0]], gbuf_a, sem_ga, add=True)
        drain_p(A)

        def dot_a(t, carry2):
            t16 = pl.ds(t * 16, 16)
            dbuf_a[t16] = (pos_a[0][t16] * pos_a[3][t16]
                           + pos_a[1][t16] * pos_a[4][t16]
                           + pos_a[2][t16] * pos_a[5][t16])
            return carry2

        lax.fori_loop(0, C // 16, dot_a, 0)
        drain_g(A)
        load_idx(ga + 2, A)
        pltpu.async_copy(gbuf_a, g_h.at[pl.ds(base_w + ga * C, C)], sem_wa)
        pltpu.async_copy(dbuf_a, dots_h.at[pl.ds(base_w + ga * C, C)], sem_wa)

        drain_idx(A)
        drain_w(A)
        front(ga + 2, A)
        drain_g(B)
        pltpu.async_copy(tb_h.at[idx_b[0]], gbuf_b, sem_gb, add=True)
        drain_p(B)

        def dot_b(t, carry2):
            t16 = pl.ds(t * 16, 16)
            dbuf_b[t16] = (pos_b[0][t16] * pos_b[3][t16]
                           + pos_b[1][t16] * pos_b[4][t16]
                           + pos_b[2][t16] * pos_b[5][t16])
            return carry2

        lax.fori_loop(0, C // 16, dot_b, 0)
        drain_g(B)

        @pl.when(o < (KCH - 1) // 2 - 1)
        def _():
            load_idx(ga + 3, B)

        pltpu.async_copy(gbuf_b, g_h.at[pl.ds(base_w + (ga + 1) * C, C)],
                         sem_wb)
        pltpu.async_copy(dbuf_b, dots_h.at[pl.ds(base_w + (ga + 1) * C, C)],
                         sem_wb)
        return carry

    lax.fori_loop(0, (KCH - 1) // 2, body, 0)

    g_last = KCH - 1
    drain_g(A)
    pltpu.async_copy(tb_h.at[idx_a[0]], gbuf_a, sem_ga, add=True)
    drain_p(A)

    def dot_tail(t, carry2):
        t16 = pl.ds(t * 16, 16)
        dbuf_a[t16] = (pos_a[0][t16] * pos_a[3][t16]
                       + pos_a[1][t16] * pos_a[4][t16]
                       + pos_a[2][t16] * pos_a[5][t16])
        return carry2

    lax.fori_loop(0, C // 16, dot_tail, 0)
    drain_g(A)
    drain_w(B)
    pltpu.async_copy(gbuf_a, g_h.at[pl.ds(base_w + g_last * C, C)], sem_wa)
    pltpu.async_copy(dbuf_a, dots_h.at[pl.ds(base_w + g_last * C, C)], sem_wa)
    drain_w(A)


_gather_call = functools.partial(
    pl.kernel,
    out_type=[
        jax.ShapeDtypeStruct((E, H), jnp.float32),
        jax.ShapeDtypeStruct((E,), jnp.float32),
    ],
    mesh=_mesh,
    scratch_types=[
        [pltpu.VMEM((C,), jnp.int32)] * 2,
        [pltpu.VMEM((C,), jnp.int32)] * 2,
        pltpu.VMEM((C, H), jnp.float32),
        pltpu.VMEM((C, H), jnp.float32),
        [pltpu.VMEM((C,), jnp.float32)] * 6,
        [pltpu.VMEM((C,), jnp.float32)] * 6,
        pltpu.VMEM((C,), jnp.float32),
        pltpu.VMEM((C,), jnp.float32),
        pltpu.SemaphoreType.DMA,
        pltpu.SemaphoreType.DMA,
        pltpu.SemaphoreType.DMA,
        pltpu.SemaphoreType.DMA,
        pltpu.SemaphoreType.DMA,
        pltpu.SemaphoreType.DMA,
        pltpu.SemaphoreType.DMA,
        pltpu.SemaphoreType.DMA,
    ],
)(_gather_body)


_BE = 1000


def _edge_mlp_body(g_ref, ea_ref, dots_ref, w1c_ref, w2_ref, b1_ref,
                   b2_ref, w1d_ref, out_ref):
    d = jnp.transpose(dots_ref[...].reshape(1, _BE), (1, 0))
    pre = (g_ref[...]
           + jnp.dot(ea_ref[...], w1c_ref[...], preferred_element_type=jnp.float32)
           + b1_ref[...]
           - 2.0 * d * w1d_ref[...])
    m1 = jnp.maximum(pre, 0.0)
    m2 = jnp.dot(m1, w2_ref[...], preferred_element_type=jnp.float32) + b2_ref[...]
    out_ref[...] = jnp.maximum(m2, 0.0)


_edge_mlp_call = pl.pallas_call(
    _edge_mlp_body,
    grid=(E // _BE,),
    in_specs=[
        pl.BlockSpec((_BE, H), lambda i: (i, 0)),
        pl.BlockSpec((_BE, DE), lambda i: (i, 0)),
        pl.BlockSpec((1, 1, _BE), lambda i: (i, 0, 0)),
        pl.BlockSpec((DE, H), lambda i: (0, 0)),
        pl.BlockSpec((H, H), lambda i: (0, 0)),
        pl.BlockSpec((1, H), lambda i: (0, 0)),
        pl.BlockSpec((1, H), lambda i: (0, 0)),
        pl.BlockSpec((1, H), lambda i: (0, 0)),
    ],
    out_specs=pl.BlockSpec((_BE, H), lambda i: (i, 0)),
    out_shape=jax.ShapeDtypeStruct((E, H), jnp.float32),
)


def _scatter_body(m2_h, dst_h, zeros_h, out_h, idx0, idx1, mb0, mb1, aggr_sp,
                  sem_i, sem_m):
    c = lax.axis_index("c")
    s = lax.axis_index("s")
    row0 = s * RPT
    pltpu.sync_copy(zeros_h.at[pl.ds(row0, RPT)],
                    aggr_sp.at[pl.ds(row0, RPT)])
    base = (c * NS + s) * EW
    plsc.subcore_barrier()

    def start(g, idxb, mbuf):
        off = base + g * CS
        pltpu.async_copy(dst_h.at[pl.ds(off, CS)], idxb, sem_i)
        pltpu.async_copy(m2_h.at[pl.ds(off, CS)], mbuf, sem_m)

    def drain():
        pltpu.make_async_copy(dst_h.at[pl.ds(base, CS)], idx0, sem_i).wait()
        pltpu.make_async_copy(m2_h.at[pl.ds(base, CS)], mb0, sem_m).wait()

    def scatter(idxb, mbuf):
        pltpu.sync_copy(mbuf, aggr_sp.at[idxb], add=True)

    start(0, idx0, mb0)

    def outer(o, carry):
        g0 = 2 * o
        drain()
        start(g0 + 1, idx1, mb1)
        scatter(idx0, mb0)
        drain()
        start(g0 + 2, idx0, mb0)
        scatter(idx1, mb1)
        return carry

    lax.fori_loop(0, (KS - 1) // 2, outer, 0)
    drain()
    scatter(idx0, mb0)
    plsc.subcore_barrier()
    pltpu.sync_copy(aggr_sp.at[pl.ds(row0, RPT)],
                    out_h.at[c, pl.ds(row0, RPT)])


_scatter_call = functools.partial(
    pl.kernel,
    out_type=jax.ShapeDtypeStruct((NC, N_PAD, H), jnp.float32),
    mesh=_mesh,
    scratch_types=[
        pltpu.VMEM((CS,), jnp.int32),
        pltpu.VMEM((CS,), jnp.int32),
        pltpu.VMEM((CS, H), jnp.float32),
        pltpu.VMEM((CS, H), jnp.float32),
        pltpu.VMEM_SHARED((N_PAD, H), jnp.float32),
        pltpu.SemaphoreType.DMA,
        pltpu.SemaphoreType.DMA,
    ],
)(_scatter_body)


def _node_mlp_body(x_ref, p_ref, wn1a_ref, wn1b_ref, bn1_ref, wn2_ref, bn2_ref,
                   out_ref):
    aggr = p_ref[0] + p_ref[1]
    h = jnp.maximum(
        jnp.dot(x_ref[...], wn1a_ref[...], preferred_element_type=jnp.float32)
        + jnp.dot(aggr, wn1b_ref[...], preferred_element_type=jnp.float32)
        + bn1_ref[...], 0.0)
    out_ref[...] = (jnp.dot(h, wn2_ref[...], preferred_element_type=jnp.float32)
                    + bn2_ref[...])


_node_mlp_call = pl.pallas_call(
    _node_mlp_body,
    grid=(N // _BN,),
    in_specs=[
        pl.BlockSpec((_BN, D), lambda i: (i, 0)),
        pl.BlockSpec((NC, _BN, H), lambda i: (0, i, 0)),
        pl.BlockSpec((D, H), lambda i: (0, 0)),
        pl.BlockSpec((H, H), lambda i: (0, 0)),
        pl.BlockSpec((1, H), lambda i: (0, 0)),
        pl.BlockSpec((H, D), lambda i: (0, 0)),
        pl.BlockSpec((1, D), lambda i: (0, 0)),
    ],
    out_specs=pl.BlockSpec((_BN, D), lambda i: (i, 0)),
    out_shape=jax.ShapeDtypeStruct((N, D), jnp.float32),
)


def kernel(x, pos, edge_index, edge_attr, W1, b1, W2, b2, Wn1, bn1, Wn2, bn2):
    ei = edge_index.astype(jnp.int32)
    dst = ei[1]
    pos4 = jnp.pad(pos, ((0, 0), (0, 13)))
    px, py, pz = pos[:, 0], pos[:, 1], pos[:, 2]
    W1a = W1[:D]
    W1b = W1[D:2 * D]
    W1c = W1[2 * D:2 * D + DE]
    w1d = W1[2 * D + DE][None, :]

    Ta, Tb = _prep_call(x, pos4, W1a, W1b, w1d)
    G, dots = _gather_call(Ta, Tb, px, py, pz, ei[0], ei[1])
    dots3d = dots.reshape(E // _BE, 1, _BE)
    m2 = _edge_mlp_call(G, edge_attr, dots3d, W1c, W2, b1[None], b2[None], w1d)
    partials = _scatter_call(m2, dst, jnp.zeros((N_PAD, H), jnp.float32))
    out = _node_mlp_call(x, partials, Wn1[:D], Wn1[D:], bn1[None], Wn2, bn2[None])
    return out
